# baseline (speedup 1.0000x reference)
import functools

import jax
import jax.numpy as jnp
from jax import lax
from jax.experimental import pallas as pl
from jax.experimental.pallas import tpu as pltpu
from jax.experimental.pallas import tpu_sc as plsc

N_USERS = 25000
N = 50000
NP = 50048
D2 = 32
E = 800000
NC = 2
NS = 16
SUB = 128
B = 1024
HB = 256
NSUB = B // SUB
NH = B // HB
EP = -(-E // (NS * B)) * B
E_PAD = EP * NS
M = E_PAD // SUB
MROWS_PER_TILE = EP // SUB
NCH = EP // B
RPT = NP // NS
CB = 136
NMC = RPT // CB


_GD = lax.GatherDimensionNumbers(offset_dims=(), collapsed_slice_dims=(0,),
                                 start_index_map=(0,))


def _splat(v16, j):
    return lax.gather(v16, jnp.full((16, 1), j, jnp.int32), _GD, (1,),
                      mode=lax.GatherScatterMode.PROMISE_IN_BOUNDS)


def _scale_rows(rows, val_v, h):
    def group(q, _):
        e0 = q * 16
        v16 = val_v[pl.ds(h * HB + e0, 16)]
        for j in range(16):
            sp = _splat(v16, j)
            e = e0 + j
            a = rows[e, pl.ds(0, 16)]
            rows[e, pl.ds(0, 16)] = a * sp
            b = rows[e, pl.ds(16, 16)]
            rows[e, pl.ds(16, 16)] = b * sp
        return 0
    lax.fori_loop(0, HB // 16, group, 0, unroll=2)


_mesh = plsc.VectorSubcoreMesh(core_axis_name="c", subcore_axis_name="s")


@functools.partial(
    pl.kernel,
    out_type=(
        jax.ShapeDtypeStruct((2 * NP, D2), jnp.float32),
        jax.ShapeDtypeStruct((2 * NP, D2), jnp.float32),
        jax.ShapeDtypeStruct((2 * NP, D2), jnp.float32),
    ),
    mesh=_mesh,
    compiler_params=pltpu.CompilerParams(use_tc_tiling_on_sc=False),
    scratch_types=[
        pltpu.VMEM((2, NSUB, SUB), jnp.int32),
        pltpu.VMEM((2, NSUB, SUB), jnp.int32),
        pltpu.VMEM((2, B), jnp.float32),
        pltpu.VMEM((HB, D2), jnp.float32),
        pltpu.VMEM((HB, D2), jnp.float32),
        pltpu.VMEM_SHARED((NP, D2), jnp.float32),
        pltpu.SemaphoreType.DMA,
        pltpu.SemaphoreType.DMA,
        pltpu.SemaphoreType.DMA,
        pltpu.SemaphoreType.DMA,
        pltpu.SemaphoreType.DMA,
        pltpu.SemaphoreType.DMA,
    ],
)
def _lightgcn_sc(x0, src2, dst2, val2, out, x1, x2,
                 src_v, dst_v, val_v, rows0, rows1, acc,
                 sem_g0, sem_g1, sem_s0, sem_s1, sem_i0, sem_i1):
    c = lax.axis_index("c")
    s = lax.axis_index("s")
    rows = (rows0, rows1)
    sems_g = (sem_g0, sem_g1)
    sems_s = (sem_s0, sem_s1)
    sems_i = (sem_i0, sem_i1)
    coff = c * NP
    z16 = jnp.zeros((16,), jnp.float32)

    def fill_zeros(i, _):
        rows0[i, pl.ds(0, 16)] = z16
        rows0[i, pl.ds(16, 16)] = z16
        return 0

    def zero_acc(k, _):
        pltpu.sync_copy(rows0.at[pl.ds(0, CB)],
                        acc.at[pl.ds(s * RPT + k * CB, CB)])
        return 0

    lax.fori_loop(0, CB, fill_zeros, 0)
    lax.fori_loop(0, NMC, zero_acc, 0)
    plsc.subcore_barrier()

    def drain_scatters():
        pltpu.make_async_copy(x0.at[pl.ds(0, HB)], rows0, sem_s0).wait()
        pltpu.make_async_copy(x0.at[pl.ds(0, HB)], rows1, sem_s1).wait()

    mbase = s * MROWS_PER_TILE

    def prefetch_idx(slot, q):
        r0 = mbase + q * NSUB
        pltpu.async_copy(src2.at[pl.ds(r0, NSUB)], src_v.at[slot],
                         sems_i[slot])
        pltpu.async_copy(dst2.at[pl.ds(r0, NSUB)], dst_v.at[slot],
                         sems_i[slot])
        pltpu.async_copy(val2.at[pl.ds(r0 * SUB, B)], val_v.at[slot],
                         sems_i[slot])

    def drain_idx(slot):
        pltpu.make_async_copy(src2.at[pl.ds(0, NSUB)], src_v.at[slot],
                              sems_i[slot]).wait()
        pltpu.make_async_copy(dst2.at[pl.ds(0, NSUB)], dst_v.at[slot],
                              sems_i[slot]).wait()
        pltpu.make_async_copy(val2.at[pl.ds(0, B)], val_v.at[slot],
                              sems_i[slot]).wait()

    def adjust_src(slot):
        for r in range(NSUB):
            for k in range(SUB // 16):
                sl = src_v[slot, r, pl.ds(k * 16, 16)]
                src_v[slot, r, pl.ds(k * 16, 16)] = sl + coff
        return

    def do_block(tab, q, a, first):
        if not first:
            drain_scatters()
        prefetched = pl.when(q + 1 < NCH)(lambda: prefetch_idx(1 - a, q + 1))
        del prefetched
        if not first:
            drain_idx(a)
        adjust_src(a)

        def gath(h, p):
            return [
                pltpu.async_copy(tab.at[src_v.at[a, 2 * h + j]],
                                 rows[p].at[pl.ds(j * SUB, SUB)],
                                 sems_g[p])
                for j in range(2)
            ]

        g_desc = {0: gath(0, 0)}
        sc_desc = {}
        for h in range(NH):
            p = h & 1
            if h + 1 < NH:
                if h >= 1:
                    for d in sc_desc[h - 1]:
                        d.wait()
                g_desc[h + 1] = gath(h + 1, 1 - p)
            for d in g_desc[h]:
                d.wait()
            _scale_rows(rows[p], val_v.at[a], h)
            sc_desc[h] = [
                pltpu.async_copy(rows[p].at[pl.ds(j * SUB, SUB)],
                                 acc.at[dst_v.at[a, 2 * h + j]],
                                 sems_s[p], add=True)
                for j in range(2)
            ]

    def do_layer(tab):
        r0 = mbase
        pltpu.sync_copy(src2.at[pl.ds(r0, NSUB)], src_v.at[0])
        pltpu.sync_copy(dst2.at[pl.ds(r0, NSUB)], dst_v.at[0])
        pltpu.sync_copy(val2.at[pl.ds(r0 * SUB, B)], val_v.at[0])
        do_block(tab, 0, 0, True)

        def pair(t, _):
            do_block(tab, 2 * t + 1, 1, False)
            do_block(tab, 2 * t + 2, 0, False)
            return 0
        lax.fori_loop(0, (NCH - 1) // 2, pair, 0)
        drain_scatters()
        plsc.subcore_barrier()

    def flush(dst_hbm):
        pltpu.sync_copy(acc.at[pl.ds(s * RPT, RPT)],
                        dst_hbm.at[pl.ds(c * NP + s * RPT, RPT)])
        lax.fori_loop(0, CB, fill_zeros, 0)
        lax.fori_loop(0, NMC, zero_acc, 0)
        plsc.subcore_barrier()

    do_layer(x0)
    flush(x1)
    do_layer(x1)
    flush(x2)
    do_layer(x2)

    base = c * NP + s * RPT

    def addrows(i, _):
        for o in (0, 16):
            rows0[i, pl.ds(o, 16)] = (rows0[i, pl.ds(o, 16)]
                                      + rows1[i, pl.ds(o, 16)])
        return 0

    def finrows(i, _):
        for o in (0, 16):
            rows0[i, pl.ds(o, 16)] = (rows0[i, pl.ds(o, 16)]
                                      + rows1[i, pl.ds(o, 16)]) * 0.25
        return 0

    def mchunk(k, _):
        r0 = base + k * CB
        a0 = s * RPT + k * CB
        av = rows0.at[pl.ds(0, CB)]
        tv = rows1.at[pl.ds(0, CB)]
        pltpu.sync_copy(x0.at[pl.ds(r0, CB)], av)
        pltpu.sync_copy(x1.at[pl.ds(r0, CB)], tv)
        lax.fori_loop(0, CB, addrows, 0)
        pltpu.sync_copy(x2.at[pl.ds(r0, CB)], tv)
        lax.fori_loop(0, CB, addrows, 0)
        pltpu.sync_copy(acc.at[pl.ds(a0, CB)], tv)
        lax.fori_loop(0, CB, finrows, 0)
        pltpu.sync_copy(av, out.at[pl.ds(r0, CB)])
        return 0
    lax.fori_loop(0, NMC, mchunk, 0)


def kernel(user_emb, item_emb, edge_val, edge_src, edge_dst):
    full = jnp.concatenate([user_emb, item_emb], axis=0)
    rpad = jnp.zeros((NP - N, D2), jnp.float32)
    x0 = jnp.concatenate([full[:, :D2], rpad, full[:, D2:], rpad],
                         axis=0)
    pad = E_PAD - E
    spread = (jnp.arange(pad, dtype=jnp.int32) * 97) % N
    srcp = jnp.concatenate([edge_src, spread]).reshape(M, SUB)
    dstp = jnp.concatenate([edge_dst, spread]).reshape(M, SUB)
    valp = jnp.concatenate([edge_val, jnp.zeros((pad,), jnp.float32)])
    out, _, _ = _lightgcn_sc(x0, srcp, dstp, valp)
    res = jnp.concatenate([out[:N], out[NP:NP + N]], axis=1)
    return (res[:N_USERS], res[N_USERS:])


# speedup vs baseline: 1.7331x; 1.7331x over previous
import functools

import jax
import jax.numpy as jnp
from jax import lax
from jax.experimental import pallas as pl
from jax.experimental.pallas import tpu as pltpu
from jax.experimental.pallas import tpu_sc as plsc

N_USERS = 25000
N = 50000
NP = 50048
D2 = 32
E = 800000
NC = 2
NS = 16
SUB = 128
B = 2048
HB = 128
NSUB = B // SUB
NH = B // HB
NBUF = 3
EP = -(-E // (NS * B)) * B
E_PAD = EP * NS
M = E_PAD // SUB
MROWS_PER_TILE = EP // SUB
NCH = EP // B
RPT = NP // NS
CB = 136
NMC = RPT // CB


_GD = lax.GatherDimensionNumbers(offset_dims=(), collapsed_slice_dims=(0,),
                                 start_index_map=(0,))


def _splat(v16, j):
    return lax.gather(v16, jnp.full((16, 1), j, jnp.int32), _GD, (1,),
                      mode=lax.GatherScatterMode.PROMISE_IN_BOUNDS)


def _scale_rows(rows, val_v, h):
    def group(q, _):
        e0 = q * 16
        v16 = val_v[pl.ds(h * HB + e0, 16)]
        for j in range(16):
            sp = _splat(v16, j)
            e = e0 + j
            a = rows[e, pl.ds(0, 16)]
            rows[e, pl.ds(0, 16)] = a * sp
            b = rows[e, pl.ds(16, 16)]
            rows[e, pl.ds(16, 16)] = b * sp
        return 0
    lax.fori_loop(0, HB // 16, group, 0)


_mesh = plsc.VectorSubcoreMesh(core_axis_name="c", subcore_axis_name="s")


@functools.partial(
    pl.kernel,
    out_type=(
        jax.ShapeDtypeStruct((2 * NP, D2), jnp.float32),
        jax.ShapeDtypeStruct((2 * NP, D2), jnp.float32),
        jax.ShapeDtypeStruct((2 * NP, D2), jnp.float32),
    ),
    mesh=_mesh,
    compiler_params=pltpu.CompilerParams(use_tc_tiling_on_sc=False),
    scratch_types=[
        pltpu.VMEM((NSUB, SUB), jnp.int32),
        pltpu.VMEM((NSUB, SUB), jnp.int32),
        pltpu.VMEM((B,), jnp.float32),
        pltpu.VMEM((HB, D2), jnp.float32),
        pltpu.VMEM((HB, D2), jnp.float32),
        pltpu.VMEM((HB, D2), jnp.float32),
        pltpu.VMEM((CB, D2), jnp.float32),
        pltpu.VMEM((CB, D2), jnp.float32),
        pltpu.VMEM_SHARED((NP, D2), jnp.float32),
        pltpu.SemaphoreType.DMA,
        pltpu.SemaphoreType.DMA,
        pltpu.SemaphoreType.DMA,
        pltpu.SemaphoreType.DMA,
        pltpu.SemaphoreType.DMA,
        pltpu.SemaphoreType.DMA,
    ],
)
def _lightgcn_sc(x0, src2, dst2, val2, out, x1, x2,
                 src_v, dst_v, val_v, rows0, rows1, rows2, av, tv, acc,
                 sem_g0, sem_g1, sem_g2, sem_s0, sem_s1, sem_s2):
    c = lax.axis_index("c")
    s = lax.axis_index("s")
    srcc = src2.at[c]
    rows = (rows0, rows1, rows2)
    sems_g = (sem_g0, sem_g1, sem_g2)
    sems_s = (sem_s0, sem_s1, sem_s2)
    z16 = jnp.zeros((16,), jnp.float32)

    def fill_zeros(i, _):
        av[i, pl.ds(0, 16)] = z16
        av[i, pl.ds(16, 16)] = z16
        return 0

    def zero_acc(k, _):
        pltpu.sync_copy(av, acc.at[pl.ds(s * RPT + k * CB, CB)])
        return 0

    lax.fori_loop(0, CB, fill_zeros, 0)
    lax.fori_loop(0, NMC, zero_acc, 0)
    plsc.subcore_barrier()

    def drain_scatters():
        for p in range(NBUF):
            pltpu.make_async_copy(x0.at[pl.ds(0, HB)], rows[p],
                                  sems_s[p]).wait()

    def do_layer(tab):
        mbase = s * MROWS_PER_TILE

        def block(q, _):
            @pl.when(q > 0)
            def _():
                drain_scatters()
            r0 = mbase + q * NSUB
            pltpu.sync_copy(srcc.at[pl.ds(r0, NSUB)], src_v)
            pltpu.sync_copy(dst2.at[pl.ds(r0, NSUB)], dst_v)
            pltpu.sync_copy(val2.at[pl.ds(r0 * SUB, B)], val_v)

            def gath(h):
                p = h % NBUF
                return pltpu.async_copy(tab.at[src_v.at[h]], rows[p],
                                        sems_g[p])

            g_desc = {0: gath(0), 1: gath(1)}
            sc_desc = {}
            for h in range(NH):
                p = h % NBUF
                if h + 2 < NH:
                    if h >= 1:
                        sc_desc[h - 1].wait()
                    g_desc[h + 2] = gath(h + 2)
                g_desc[h].wait()
                _scale_rows(rows[p], val_v, h)
                sc_desc[h] = pltpu.async_copy(rows[p], acc.at[dst_v.at[h]],
                                              sems_s[p], add=True)
            return 0
        lax.fori_loop(0, NCH, block, 0)
        drain_scatters()
        plsc.subcore_barrier()

    def flush(dst_hbm):
        pltpu.sync_copy(acc.at[pl.ds(s * RPT, RPT)],
                        dst_hbm.at[pl.ds(c * NP + s * RPT, RPT)])
        lax.fori_loop(0, NMC, zero_acc, 0)
        plsc.subcore_barrier()

    do_layer(x0)
    flush(x1)
    do_layer(x1)
    flush(x2)
    do_layer(x2)

    base = c * NP + s * RPT

    def addrows(i, _):
        for o in (0, 16):
            av[i, pl.ds(o, 16)] = av[i, pl.ds(o, 16)] + tv[i, pl.ds(o, 16)]
        return 0

    def finrows(i, _):
        for o in (0, 16):
            av[i, pl.ds(o, 16)] = (av[i, pl.ds(o, 16)]
                                   + tv[i, pl.ds(o, 16)]) * 0.25
        return 0

    def mchunk(k, _):
        r0 = base + k * CB
        a0 = s * RPT + k * CB
        pltpu.sync_copy(x0.at[pl.ds(r0, CB)], av)
        pltpu.sync_copy(x1.at[pl.ds(r0, CB)], tv)
        lax.fori_loop(0, CB, addrows, 0)
        pltpu.sync_copy(x2.at[pl.ds(r0, CB)], tv)
        lax.fori_loop(0, CB, addrows, 0)
        pltpu.sync_copy(acc.at[pl.ds(a0, CB)], tv)
        lax.fori_loop(0, CB, finrows, 0)
        pltpu.sync_copy(av, out.at[pl.ds(r0, CB)])
        return 0
    lax.fori_loop(0, NMC, mchunk, 0)


def kernel(user_emb, item_emb, edge_val, edge_src, edge_dst):
    full = jnp.concatenate([user_emb, item_emb], axis=0)
    rpad = jnp.zeros((NP - N, D2), jnp.float32)
    x0 = jnp.concatenate([full[:, :D2], rpad, full[:, D2:], rpad],
                         axis=0)
    pad = E_PAD - E
    spread = (jnp.arange(pad, dtype=jnp.int32) * 97) % N
    srcp = jnp.concatenate([edge_src, spread])
    dstp = jnp.concatenate([edge_dst, spread])
    valp = jnp.concatenate([edge_val, jnp.zeros((pad,), jnp.float32)])
    src2 = jnp.stack([srcp, srcp + NP]).reshape(2, M, SUB)
    dst2 = dstp.reshape(M, SUB)
    out, _, _ = _lightgcn_sc(x0, src2, dst2, valp)
    res = jnp.concatenate([out[:N], out[NP:NP + N]], axis=1)
    return (res[:N_USERS], res[N_USERS:])


# speedup vs baseline: 1.8714x; 1.0798x over previous
import functools

import jax
import jax.numpy as jnp
from jax import lax
from jax.experimental import pallas as pl
from jax.experimental.pallas import tpu as pltpu
from jax.experimental.pallas import tpu_sc as plsc

N_USERS = 25000
N = 50000
NP = 50048
D2 = 32
E = 800000
NC = 2
NS = 16
SUB = 128
B = 2048
HB = 128
NSUB = B // SUB
NH = B // HB
NBUF = 3
EP = -(-E // (NS * B)) * B
E_PAD = EP * NS
M = E_PAD // SUB
MROWS_PER_TILE = EP // SUB
NCH = EP // B
RPT = NP // NS
CB = 136
NMC = RPT // CB


_GD = lax.GatherDimensionNumbers(offset_dims=(), collapsed_slice_dims=(0,),
                                 start_index_map=(0,))


def _splat(v16, j):
    return lax.gather(v16, jnp.full((16, 1), j, jnp.int32), _GD, (1,),
                      mode=lax.GatherScatterMode.PROMISE_IN_BOUNDS)


def _scale_rows(rows, val_v, h):
    def group(q, _):
        e0 = q * 16
        v16 = val_v[pl.ds(h * HB + e0, 16)]
        for j in range(16):
            sp = _splat(v16, j)
            e = e0 + j
            a = rows[e, pl.ds(0, 16)]
            rows[e, pl.ds(0, 16)] = a * sp
            b = rows[e, pl.ds(16, 16)]
            rows[e, pl.ds(16, 16)] = b * sp
        return 0
    lax.fori_loop(0, HB // 16, group, 0)


_mesh = plsc.VectorSubcoreMesh(core_axis_name="c", subcore_axis_name="s")


@functools.partial(
    pl.kernel,
    out_type=(
        jax.ShapeDtypeStruct((2 * NP, D2), jnp.float32),
        jax.ShapeDtypeStruct((2 * NP, D2), jnp.float32),
        jax.ShapeDtypeStruct((2 * NP, D2), jnp.float32),
    ),
    mesh=_mesh,
    compiler_params=pltpu.CompilerParams(use_tc_tiling_on_sc=False),
    scratch_types=[
        pltpu.VMEM((NSUB, SUB), jnp.int32),
        pltpu.VMEM((NSUB, SUB), jnp.int32),
        pltpu.VMEM((B,), jnp.float32),
        pltpu.VMEM((HB, D2), jnp.float32),
        pltpu.VMEM((HB, D2), jnp.float32),
        pltpu.VMEM((HB, D2), jnp.float32),
        pltpu.VMEM((CB, D2), jnp.float32),
        pltpu.VMEM((CB, D2), jnp.float32),
        pltpu.VMEM_SHARED((NP, D2), jnp.float32),
        pltpu.SemaphoreType.DMA,
        pltpu.SemaphoreType.DMA,
        pltpu.SemaphoreType.DMA,
        pltpu.SemaphoreType.DMA,
        pltpu.SemaphoreType.DMA,
        pltpu.SemaphoreType.DMA,
        pltpu.SemaphoreType.DMA,
    ],
)
def _lightgcn_sc(x0, src2, dst2, val2, out, x1, x2,
                 src_v, dst_v, val_v, rows0, rows1, rows2, av, tv, acc,
                 sem_g0, sem_g1, sem_g2, sem_s0, sem_s1, sem_s2, sem_i):
    c = lax.axis_index("c")
    s = lax.axis_index("s")
    srcc = src2.at[c]
    rows = (rows0, rows1, rows2)
    sems_g = (sem_g0, sem_g1, sem_g2)
    sems_s = (sem_s0, sem_s1, sem_s2)
    z16 = jnp.zeros((16,), jnp.float32)

    def fill_zeros(i, _):
        av[i, pl.ds(0, 16)] = z16
        av[i, pl.ds(16, 16)] = z16
        return 0

    def zero_acc():
        def zk(k, _):
            pltpu.sync_copy(av, acc.at[pl.ds(s * RPT + k * CB, CB)])
            return 0
        lax.fori_loop(0, NMC, zk, 0)

    lax.fori_loop(0, CB, fill_zeros, 0)
    zero_acc()
    plsc.subcore_barrier()

    def drain_scatters():
        for p in range(NBUF):
            pltpu.make_async_copy(x0.at[pl.ds(0, HB)], rows[p],
                                  sems_s[p]).wait()

    def do_layer(tab):
        mbase = s * MROWS_PER_TILE

        def block(q, _):
            @pl.when(q > 0)
            def _():
                drain_scatters()
            r0 = mbase + q * NSUB
            idx_cps = [
                pltpu.async_copy(srcc.at[pl.ds(r0, NSUB)], src_v, sem_i),
                pltpu.async_copy(dst2.at[pl.ds(r0, NSUB)], dst_v, sem_i),
                pltpu.async_copy(val2.at[pl.ds(r0 * SUB, B)], val_v, sem_i),
            ]
            for d in idx_cps:
                d.wait()

            def gath(h):
                p = h % NBUF
                return pltpu.async_copy(tab.at[src_v.at[h]], rows[p],
                                        sems_g[p])

            g_desc = {0: gath(0), 1: gath(1)}
            sc_desc = {}
            for h in range(NH):
                p = h % NBUF
                if h + 2 < NH:
                    if h >= 1:
                        sc_desc[h - 1].wait()
                    g_desc[h + 2] = gath(h + 2)
                g_desc[h].wait()
                _scale_rows(rows[p], val_v, h)
                sc_desc[h] = pltpu.async_copy(rows[p], acc.at[dst_v.at[h]],
                                              sems_s[p], add=True)
            return 0
        lax.fori_loop(0, NCH, block, 0)
        drain_scatters()
        plsc.subcore_barrier()

    def flush(dst_hbm):
        pltpu.sync_copy(acc.at[pl.ds(s * RPT, RPT)],
                        dst_hbm.at[pl.ds(c * NP + s * RPT, RPT)])
        zero_acc()
        plsc.subcore_barrier()

    do_layer(x0)
    flush(x1)
    do_layer(x1)
    flush(x2)
    do_layer(x2)

    base = c * NP + s * RPT

    def addrows(i, _):
        for o in (0, 16):
            av[i, pl.ds(o, 16)] = av[i, pl.ds(o, 16)] + tv[i, pl.ds(o, 16)]
        return 0

    def finrows(i, _):
        for o in (0, 16):
            av[i, pl.ds(o, 16)] = (av[i, pl.ds(o, 16)]
                                   + tv[i, pl.ds(o, 16)]) * 0.25
        return 0

    def mchunk(k, _):
        r0 = base + k * CB
        a0 = s * RPT + k * CB
        pltpu.sync_copy(x0.at[pl.ds(r0, CB)], av)
        pltpu.sync_copy(x1.at[pl.ds(r0, CB)], tv)
        lax.fori_loop(0, CB, addrows, 0)
        pltpu.sync_copy(x2.at[pl.ds(r0, CB)], tv)
        lax.fori_loop(0, CB, addrows, 0)
        pltpu.sync_copy(acc.at[pl.ds(a0, CB)], tv)
        lax.fori_loop(0, CB, finrows, 0)
        pltpu.sync_copy(av, out.at[pl.ds(r0, CB)])
        return 0
    lax.fori_loop(0, NMC, mchunk, 0)


def kernel(user_emb, item_emb, edge_val, edge_src, edge_dst):
    full = jnp.concatenate([user_emb, item_emb], axis=0)
    rpad = jnp.zeros((NP - N, D2), jnp.float32)
    x0 = jnp.concatenate([full[:, :D2], rpad, full[:, D2:], rpad],
                         axis=0)
    pad = E_PAD - E
    spread = (jnp.arange(pad, dtype=jnp.int32) * 97) % N
    srcp = jnp.concatenate([edge_src, spread])
    dstp = jnp.concatenate([edge_dst, spread])
    valp = jnp.concatenate([edge_val, jnp.zeros((pad,), jnp.float32)])
    src2 = jnp.stack([srcp, srcp + NP]).reshape(2, M, SUB)
    dst2 = dstp.reshape(M, SUB)
    out, _, _ = _lightgcn_sc(x0, src2, dst2, valp)
    res = jnp.concatenate([out[:N], out[NP:NP + N]], axis=1)
    return (res[:N_USERS], res[N_USERS:])


# speedup vs baseline: 1.9028x; 1.0168x over previous
import functools

import jax
import jax.numpy as jnp
from jax import lax
from jax.experimental import pallas as pl
from jax.experimental.pallas import tpu as pltpu
from jax.experimental.pallas import tpu_sc as plsc

N_USERS = 25000
N = 50000
NP = 50048
D2 = 32
E = 800000
NC = 2
NS = 16
SUB = 128
B = 2048
HB = 128
NSUB = B // SUB
NH = B // HB
NBUF = 3
EP = -(-E // (NS * B)) * B
E_PAD = EP * NS
M = E_PAD // SUB
MROWS_PER_TILE = EP // SUB
NCH = EP // B
RPT = NP // NS
CB = 136
NMC = RPT // CB


_GD = lax.GatherDimensionNumbers(offset_dims=(), collapsed_slice_dims=(0,),
                                 start_index_map=(0,))


def _splat(v16, j):
    return lax.gather(v16, jnp.full((16, 1), j, jnp.int32), _GD, (1,),
                      mode=lax.GatherScatterMode.PROMISE_IN_BOUNDS)


def _scale_rows(rows, val_v, h):
    def group(q, _):
        e0 = q * 16
        v16 = val_v[pl.ds(h * HB + e0, 16)]
        for j in range(16):
            sp = _splat(v16, j)
            e = e0 + j
            a = rows[e, pl.ds(0, 16)]
            rows[e, pl.ds(0, 16)] = a * sp
            b = rows[e, pl.ds(16, 16)]
            rows[e, pl.ds(16, 16)] = b * sp
        return 0
    lax.fori_loop(0, HB // 16, group, 0)


_mesh = plsc.VectorSubcoreMesh(core_axis_name="c", subcore_axis_name="s")


@functools.partial(
    pl.kernel,
    out_type=(
        jax.ShapeDtypeStruct((2 * NP, D2), jnp.float32),
        jax.ShapeDtypeStruct((2 * NP, D2), jnp.float32),
        jax.ShapeDtypeStruct((2 * NP, D2), jnp.float32),
    ),
    mesh=_mesh,
    compiler_params=pltpu.CompilerParams(use_tc_tiling_on_sc=False),
    scratch_types=[
        pltpu.VMEM((NSUB, SUB), jnp.int32),
        pltpu.VMEM((NSUB, SUB), jnp.int32),
        pltpu.VMEM((B,), jnp.float32),
        pltpu.VMEM((HB, D2), jnp.float32),
        pltpu.VMEM((HB, D2), jnp.float32),
        pltpu.VMEM((HB, D2), jnp.float32),
        pltpu.VMEM((CB, D2), jnp.float32),
        pltpu.VMEM((CB, D2), jnp.float32),
        pltpu.VMEM_SHARED((NP, D2), jnp.float32),
        pltpu.SemaphoreType.DMA,
        pltpu.SemaphoreType.DMA,
        pltpu.SemaphoreType.DMA,
        pltpu.SemaphoreType.DMA,
        pltpu.SemaphoreType.DMA,
        pltpu.SemaphoreType.DMA,
        pltpu.SemaphoreType.DMA,
    ],
)
def _lightgcn_sc(x0, src2, dst2, val2, out, x1, x2,
                 src_v, dst_v, val_v, rows0, rows1, rows2, av, tv, acc,
                 sem_g0, sem_g1, sem_g2, sem_s0, sem_s1, sem_s2, sem_i):
    c = lax.axis_index("c")
    s = lax.axis_index("s")
    srcc = src2.at[c]
    rows = (rows0, rows1, rows2)
    sems_g = (sem_g0, sem_g1, sem_g2)
    sems_s = (sem_s0, sem_s1, sem_s2)
    z16 = jnp.zeros((16,), jnp.float32)

    def fill_zeros(i, _):
        av[i, pl.ds(0, 16)] = z16
        av[i, pl.ds(16, 16)] = z16
        return 0

    def zero_acc():
        cps = [pltpu.async_copy(av, acc.at[pl.ds(s * RPT + k * CB, CB)],
                                sem_i)
               for k in range(NMC)]
        for d in cps:
            d.wait()

    lax.fori_loop(0, CB, fill_zeros, 0)
    zero_acc()
    plsc.subcore_barrier()

    def drain_scatters():
        for p in range(NBUF):
            pltpu.make_async_copy(x0.at[pl.ds(0, HB)], rows[p],
                                  sems_s[p]).wait()

    def do_layer(tab):
        mbase = s * MROWS_PER_TILE

        def block(q, _):
            @pl.when(q > 0)
            def _():
                drain_scatters()
            r0 = mbase + q * NSUB
            idx_cps = [
                pltpu.async_copy(srcc.at[pl.ds(r0, NSUB)], src_v, sem_i),
                pltpu.async_copy(dst2.at[pl.ds(r0, NSUB)], dst_v, sem_i),
                pltpu.async_copy(val2.at[pl.ds(r0 * SUB, B)], val_v, sem_i),
            ]
            for d in idx_cps:
                d.wait()

            def gath(h):
                p = h % NBUF
                return pltpu.async_copy(tab.at[src_v.at[h]], rows[p],
                                        sems_g[p])

            g_desc = {0: gath(0), 1: gath(1)}
            sc_desc = {}
            for h in range(NH):
                p = h % NBUF
                if h + 2 < NH:
                    if h >= 1:
                        sc_desc[h - 1].wait()
                    g_desc[h + 2] = gath(h + 2)
                g_desc[h].wait()
                _scale_rows(rows[p], val_v, h)
                sc_desc[h] = pltpu.async_copy(rows[p], acc.at[dst_v.at[h]],
                                              sems_s[p], add=True)
            return 0
        lax.fori_loop(0, NCH, block, 0)
        drain_scatters()
        plsc.subcore_barrier()

    def flush(dst_hbm):
        pltpu.sync_copy(acc.at[pl.ds(s * RPT, RPT)],
                        dst_hbm.at[pl.ds(c * NP + s * RPT, RPT)])
        zero_acc()
        plsc.subcore_barrier()

    do_layer(x0)
    flush(x1)
    do_layer(x1)
    flush(x2)
    do_layer(x2)

    base = c * NP + s * RPT

    def addrows(i, _):
        for o in (0, 16):
            av[i, pl.ds(o, 16)] = av[i, pl.ds(o, 16)] + tv[i, pl.ds(o, 16)]
        return 0

    def finrows(i, _):
        for o in (0, 16):
            av[i, pl.ds(o, 16)] = (av[i, pl.ds(o, 16)]
                                   + tv[i, pl.ds(o, 16)]) * 0.25
        return 0

    def mchunk(k, _):
        r0 = base + k * CB
        a0 = s * RPT + k * CB
        cps = [pltpu.async_copy(x0.at[pl.ds(r0, CB)], av, sem_i),
               pltpu.async_copy(x1.at[pl.ds(r0, CB)], tv, sem_i)]
        for d in cps:
            d.wait()
        lax.fori_loop(0, CB, addrows, 0)
        pltpu.sync_copy(x2.at[pl.ds(r0, CB)], tv)
        lax.fori_loop(0, CB, addrows, 0)
        pltpu.sync_copy(acc.at[pl.ds(a0, CB)], tv)
        lax.fori_loop(0, CB, finrows, 0)
        pltpu.sync_copy(av, out.at[pl.ds(r0, CB)])
        return 0
    lax.fori_loop(0, NMC, mchunk, 0)


def kernel(user_emb, item_emb, edge_val, edge_src, edge_dst):
    full = jnp.concatenate([user_emb, item_emb], axis=0)
    rpad = jnp.zeros((NP - N, D2), jnp.float32)
    x0 = jnp.concatenate([full[:, :D2], rpad, full[:, D2:], rpad],
                         axis=0)
    pad = E_PAD - E
    spread = (jnp.arange(pad, dtype=jnp.int32) * 97) % N
    srcp = jnp.concatenate([edge_src, spread])
    dstp = jnp.concatenate([edge_dst, spread])
    valp = jnp.concatenate([edge_val, jnp.zeros((pad,), jnp.float32)])
    src2 = jnp.stack([srcp, srcp + NP]).reshape(2, M, SUB)
    dst2 = dstp.reshape(M, SUB)
    out, _, _ = _lightgcn_sc(x0, src2, dst2, valp)
    res = jnp.concatenate([out[:N], out[NP:NP + N]], axis=1)
    return (res[:N_USERS], res[N_USERS:])
